# baseline (speedup 1.0000x reference)
import jax
import jax.numpy as jnp
from jax import lax
from jax.experimental import pallas as pl
from jax.experimental.pallas import tpu as pltpu

F32 = jnp.float32
D = 768
H = 12
DH = 64
DFF = 3072
E = 8
DEPTH = 4
PP = 16
NPATCH = 14
T = NPATCH * NPATCH + 1
NB = 8
NTOK = NB * T
NCLS = 1000
C = int(1.25 * NTOK * 2 / E)
NSLOT = E * C


def _ln(x, s, b):
    m = jnp.mean(x, axis=-1, keepdims=True)
    v = jnp.mean((x - m) ** 2, axis=-1, keepdims=True)
    return (x - m) / jnp.sqrt(v + 1e-6) * s + b


def _embed_kern(p_ref, w_ref, b_ref, cls_ref, pos_ref, o_ref):
    mm = jnp.dot(p_ref[0], w_ref[...], preferred_element_type=F32) + b_ref[...]
    full = jnp.concatenate([cls_ref[0], mm], axis=0)
    o_ref[0] = full + pos_ref[0]


def _embed(p, patch_w, patch_b, cls_tok, pos):
    return pl.pallas_call(
        _embed_kern,
        grid=(NB,),
        in_specs=[
            pl.BlockSpec((1, NPATCH * NPATCH, 3 * PP * PP), lambda b: (b, 0, 0)),
            pl.BlockSpec((3 * PP * PP, D), lambda b: (0, 0)),
            pl.BlockSpec((1, D), lambda b: (0, 0)),
            pl.BlockSpec((1, 1, D), lambda b: (0, 0, 0)),
            pl.BlockSpec((1, T, D), lambda b: (0, 0, 0)),
        ],
        out_specs=pl.BlockSpec((1, T, D), lambda b: (b, 0, 0)),
        out_shape=jax.ShapeDtypeStruct((NB, T, D), F32),
    )(p, patch_w, patch_b.reshape(1, D), cls_tok, pos)


def _attn_kern(h_ref, ls_ref, lb_ref, wqkv_ref, bqkv_ref, wp_ref, bp_ref, o_ref):
    x = h_ref[0]
    xn = _ln(x, ls_ref[...], lb_ref[...])
    qkv = jnp.dot(xn, wqkv_ref[...], preferred_element_type=F32) + bqkv_ref[...]
    outs = []
    for hh in range(H):
        q = qkv[:, hh * DH:(hh + 1) * DH]
        k = qkv[:, D + hh * DH:D + (hh + 1) * DH]
        v = qkv[:, 2 * D + hh * DH:2 * D + (hh + 1) * DH]
        s = lax.dot_general(q, k, (((1,), (1,)), ((), ())),
                            preferred_element_type=F32) * (DH ** -0.5)
        a = jax.nn.softmax(s, axis=-1)
        outs.append(jnp.dot(a, v, preferred_element_type=F32))
    o = jnp.concatenate(outs, axis=1)
    o_ref[0] = x + jnp.dot(o, wp_ref[...], preferred_element_type=F32) + bp_ref[...]


def _attn(h, ls, lb, wqkv, bqkv, wp, bp):
    return pl.pallas_call(
        _attn_kern,
        grid=(NB,),
        in_specs=[
            pl.BlockSpec((1, T, D), lambda b: (b, 0, 0)),
            pl.BlockSpec((1, D), lambda b: (0, 0)),
            pl.BlockSpec((1, D), lambda b: (0, 0)),
            pl.BlockSpec((D, 3 * H * DH), lambda b: (0, 0)),
            pl.BlockSpec((1, 3 * H * DH), lambda b: (0, 0)),
            pl.BlockSpec((H * DH, D), lambda b: (0, 0)),
            pl.BlockSpec((1, D), lambda b: (0, 0)),
        ],
        out_specs=pl.BlockSpec((1, T, D), lambda b: (b, 0, 0)),
        out_shape=jax.ShapeDtypeStruct((NB, T, D), F32),
    )(h, ls.reshape(1, D), lb.reshape(1, D), wqkv, bqkv.reshape(1, -1), wp,
      bp.reshape(1, D))


def _mlp_kern(h_ref, ls_ref, lb_ref, w1_ref, b1_ref, w2_ref, b2_ref, o_ref):
    x = h_ref[0]
    z = _ln(x, ls_ref[...], lb_ref[...])
    f = jax.nn.gelu(jnp.dot(z, w1_ref[...], preferred_element_type=F32) + b1_ref[...])
    o_ref[0] = x + jnp.dot(f, w2_ref[...], preferred_element_type=F32) + b2_ref[...]


def _mlp(h, ls, lb, w1, b1, w2, b2):
    return pl.pallas_call(
        _mlp_kern,
        grid=(NB,),
        in_specs=[
            pl.BlockSpec((1, T, D), lambda b: (b, 0, 0)),
            pl.BlockSpec((1, D), lambda b: (0, 0)),
            pl.BlockSpec((1, D), lambda b: (0, 0)),
            pl.BlockSpec((D, DFF), lambda b: (0, 0)),
            pl.BlockSpec((1, DFF), lambda b: (0, 0)),
            pl.BlockSpec((DFF, D), lambda b: (0, 0)),
            pl.BlockSpec((1, D), lambda b: (0, 0)),
        ],
        out_specs=pl.BlockSpec((1, T, D), lambda b: (b, 0, 0)),
        out_shape=jax.ShapeDtypeStruct((NB, T, D), F32),
    )(h, ls.reshape(1, D), lb.reshape(1, D), w1, b1.reshape(1, DFF), w2,
      b2.reshape(1, D))


def _route_kern(x_ref, ls_ref, lb_ref, wg_ref, z_ref,
                i1_ref, p1_ref, g1_ref, i2_ref, p2_ref, g2_ref):
    x = x_ref[...]
    z = _ln(x, ls_ref[...], lb_ref[...])
    z_ref[...] = z
    logits = jnp.dot(z, wg_ref[...], preferred_element_type=F32)
    gates = jax.nn.softmax(logits, axis=-1)
    eio = lax.broadcasted_iota(jnp.int32, (NTOK, E), 1)
    v1 = jnp.max(gates, axis=-1, keepdims=True)
    i1 = jnp.min(jnp.where(gates >= v1, eio, E), axis=-1, keepdims=True)
    m1 = (eio == i1).astype(F32)
    gates2 = gates - m1 * 2.0
    v2 = jnp.max(gates2, axis=-1, keepdims=True)
    i2 = jnp.min(jnp.where(gates2 >= v2, eio, E), axis=-1, keepdims=True)
    m2 = (eio == i2).astype(F32)
    rio = lax.broadcasted_iota(jnp.int32, (NTOK, NTOK), 0)
    cio = lax.broadcasted_iota(jnp.int32, (NTOK, NTOK), 1)
    ltri = (rio >= cio).astype(F32)
    loc1 = jnp.dot(ltri, m1, preferred_element_type=F32) - 1.0
    cnt1 = jnp.sum(m1, axis=0, keepdims=True)
    loc2 = jnp.dot(ltri, m2, preferred_element_type=F32) - 1.0 + cnt1
    m1k = m1 * (loc1 < C).astype(F32)
    m2k = m2 * (loc2 < C).astype(F32)
    p1 = jnp.sum(loc1 * m1k, axis=-1, keepdims=True)
    p2 = jnp.sum(loc2 * m2k, axis=-1, keepdims=True)
    k1 = jnp.sum(m1k, axis=-1, keepdims=True)
    k2 = jnp.sum(m2k, axis=-1, keepdims=True)
    den = v1 + v2 + 1e-9
    i1_ref[...] = i1.astype(F32)
    p1_ref[...] = p1
    g1_ref[...] = v1 / den * k1
    i2_ref[...] = i2.astype(F32)
    p2_ref[...] = p2
    g2_ref[...] = v2 / den * k2


def _route(xflat, ls, lb, wg):
    col = jax.ShapeDtypeStruct((NTOK, 1), F32)
    return pl.pallas_call(
        _route_kern,
        in_specs=[
            pl.BlockSpec((NTOK, D), lambda: (0, 0)),
            pl.BlockSpec((1, D), lambda: (0, 0)),
            pl.BlockSpec((1, D), lambda: (0, 0)),
            pl.BlockSpec((D, E), lambda: (0, 0)),
        ],
        out_specs=[
            pl.BlockSpec((NTOK, D), lambda: (0, 0)),
        ] + [pl.BlockSpec((NTOK, 1), lambda: (0, 0))] * 6,
        out_shape=[jax.ShapeDtypeStruct((NTOK, D), F32)] + [col] * 6,
    )(xflat, ls.reshape(1, D), lb.reshape(1, D), wg)


def _disp_kern(z_ref, r_ref, o_ref):
    e = pl.program_id(0)
    r = r_ref[...]
    i1 = r[:, 0:1].astype(jnp.int32)
    p1 = r[:, 1:2].astype(jnp.int32)
    g1 = r[:, 2:3]
    i2 = r[:, 3:4].astype(jnp.int32)
    p2 = r[:, 4:5].astype(jnp.int32)
    g2 = r[:, 5:6]
    cc = lax.broadcasted_iota(jnp.int32, (NTOK, C), 1)
    d1 = ((i1 == e) & (cc == p1)).astype(F32) * (g1 > 0).astype(F32)
    d2 = ((i2 == e) & (cc == p2)).astype(F32) * (g2 > 0).astype(F32)
    d = d1 + d2
    o_ref[0] = lax.dot_general(d, z_ref[...], (((0,), (0,)), ((), ())),
                               preferred_element_type=F32)


def _dispatch(z, r):
    return pl.pallas_call(
        _disp_kern,
        grid=(E,),
        in_specs=[
            pl.BlockSpec((NTOK, D), lambda e: (0, 0)),
            pl.BlockSpec((NTOK, 8), lambda e: (0, 0)),
        ],
        out_specs=pl.BlockSpec((1, C, D), lambda e: (e, 0, 0)),
        out_shape=jax.ShapeDtypeStruct((E, C, D), F32),
    )(z, r)


def _expert_kern(x_ref, w1_ref, b1_ref, w2_ref, b2_ref, o_ref):
    x = x_ref[0]
    hmid = jax.nn.gelu(jnp.dot(x, w1_ref[0], preferred_element_type=F32)
                       + b1_ref[0])
    o_ref[0] = jnp.dot(hmid, w2_ref[0], preferred_element_type=F32) + b2_ref[0]


def _experts(xe, w1, b1, w2, b2):
    return pl.pallas_call(
        _expert_kern,
        grid=(E,),
        in_specs=[
            pl.BlockSpec((1, C, D), lambda e: (e, 0, 0)),
            pl.BlockSpec((1, D, DFF), lambda e: (e, 0, 0)),
            pl.BlockSpec((1, 1, DFF), lambda e: (e, 0, 0)),
            pl.BlockSpec((1, DFF, D), lambda e: (e, 0, 0)),
            pl.BlockSpec((1, 1, D), lambda e: (e, 0, 0)),
        ],
        out_specs=pl.BlockSpec((1, C, D), lambda e: (e, 0, 0)),
        out_shape=jax.ShapeDtypeStruct((E, C, D), F32),
    )(xe, w1, b1.reshape(E, 1, DFF), w2, b2.reshape(E, 1, D))


def _comb_kern(h_ref, r_ref, o_ref, y_ref):
    r = r_ref[0]
    i1 = r[:, 0:1].astype(jnp.int32)
    p1 = r[:, 1:2].astype(jnp.int32)
    g1 = r[:, 2:3]
    i2 = r[:, 3:4].astype(jnp.int32)
    p2 = r[:, 4:5].astype(jnp.int32)
    g2 = r[:, 5:6]
    sio = lax.broadcasted_iota(jnp.int32, (T, NSLOT), 1)
    s1 = i1 * C + p1
    s2 = i2 * C + p2
    comb = jnp.where(sio == s1, g1, 0.0) + jnp.where(sio == s2, g2, 0.0)
    y = jnp.dot(comb, o_ref[...], preferred_element_type=F32)
    y_ref[0] = h_ref[0] + y


def _combine(h, r3, oflat):
    return pl.pallas_call(
        _comb_kern,
        grid=(NB,),
        in_specs=[
            pl.BlockSpec((1, T, D), lambda b: (b, 0, 0)),
            pl.BlockSpec((1, T, 8), lambda b: (b, 0, 0)),
            pl.BlockSpec((NSLOT, D), lambda b: (0, 0)),
        ],
        out_specs=pl.BlockSpec((1, T, D), lambda b: (b, 0, 0)),
        out_shape=jax.ShapeDtypeStruct((NB, T, D), F32),
    )(h, r3, oflat)


def _final_kern(h_ref, ls_ref, lb_ref, w_ref, b_ref, o_ref):
    xn = _ln(h_ref[...], ls_ref[...], lb_ref[...])
    m = jnp.mean(xn, axis=1)
    o_ref[...] = jnp.dot(m, w_ref[...], preferred_element_type=F32) + b_ref[...]


def _final(h, ls, lb, w, b):
    return pl.pallas_call(
        _final_kern,
        in_specs=[
            pl.BlockSpec((NB, T, D), lambda: (0, 0, 0)),
            pl.BlockSpec((1, D), lambda: (0, 0)),
            pl.BlockSpec((1, D), lambda: (0, 0)),
            pl.BlockSpec((D, NCLS), lambda: (0, 0)),
            pl.BlockSpec((1, NCLS), lambda: (0, 0)),
        ],
        out_specs=pl.BlockSpec((NB, NCLS), lambda: (0, 0)),
        out_shape=jax.ShapeDtypeStruct((NB, NCLS), F32),
    )(h, ls.reshape(1, D), lb.reshape(1, D), w, b.reshape(1, NCLS))


def kernel(x, patch_w, patch_b, cls_tok, pos, ln1_s, ln1_b, qkv_w, qkv_b,
           proj_w, proj_b, ln2_s, ln2_b, mlp_w1, mlp_b1, mlp_w2, mlp_b2,
           gate_w, moe_w1, moe_b1, moe_w2, moe_b2, lnf_s, lnf_b, cls_w, cls_b):
    p = x.reshape(NB, 3, NPATCH, PP, NPATCH, PP)
    p = p.transpose(0, 2, 4, 1, 3, 5).reshape(NB, NPATCH * NPATCH, 3 * PP * PP)
    h = _embed(p, patch_w, patch_b, cls_tok, pos)
    for i in range(DEPTH):
        h = _attn(h, ln1_s[i], ln1_b[i], qkv_w[i], qkv_b[i], proj_w[i],
                  proj_b[i])
        if i % 2 == 0:
            j = i // 2
            h = _mlp(h, ln2_s[i], ln2_b[i], mlp_w1[j], mlp_b1[j], mlp_w2[j],
                     mlp_b2[j])
        else:
            j = i // 2
            z, i1, p1, g1, i2, p2, g2 = _route(h.reshape(NTOK, D), ln2_s[i],
                                               ln2_b[i], gate_w[j])
            r = jnp.concatenate([i1, p1, g1, i2, p2, g2,
                                 jnp.zeros_like(g1), jnp.zeros_like(g2)],
                                axis=1)
            xe = _dispatch(z, r)
            o = _experts(xe, moe_w1[j], moe_b1[j], moe_w2[j], moe_b2[j])
            h = _combine(h, r.reshape(NB, T, 8), o.reshape(NSLOT, D))
    return _final(h, lnf_s, lnf_b, cls_w, cls_b)
